# Initial kernel scaffold; baseline (speedup 1.0000x reference)
import jax
import jax.numpy as jnp
from jax.experimental import pallas as pl


def kernel(x, batch, sensor_emb, Wq1, Wk1, Wv1, av1, ln1_g, ln1_b, Wq2, Wk2, Wv2, av2, ln2_g, ln2_b, Wf1, bf1, Wf2, bf2):
    raise NotImplementedError("write your pallas kernel here")


# speedup vs baseline: 433.4817x; 433.4817x over previous
import functools

import jax
import jax.numpy as jnp
from jax.experimental import pallas as pl
from jax.experimental.pallas import tpu as pltpu

NS = 2048
IN_DIM = 64
HID = 64
TOPK = 32
HEADS = 4
DH = HID // HEADS
RB = 256
NBLK = NS // RB
NEG = -1e30


def _topk_mask_kernel(emb_ref, mask_ref):
    i = pl.program_id(0)
    emb = emb_ref[...]
    nrm = jnp.sqrt(jnp.sum(emb * emb, axis=1, keepdims=True))
    embn = emb / jnp.maximum(nrm, 1e-12)
    rows_raw = emb_ref[pl.ds(i * RB, RB), :]
    rnrm = jnp.sqrt(jnp.sum(rows_raw * rows_raw, axis=1, keepdims=True))
    rows = rows_raw / jnp.maximum(rnrm, 1e-12)
    sim = jax.lax.dot_general(rows, embn, (((1,), (1,)), ((), ())),
                              preferred_element_type=jnp.float32)
    col = jax.lax.broadcasted_iota(jnp.int32, (RB, NS), 1)
    row = jax.lax.broadcasted_iota(jnp.int32, (RB, NS), 0)
    sim = jnp.where(col == row + i * RB, -1e9, sim)

    def step(_, carry):
        val, mask = carry
        m = jnp.max(val, axis=1, keepdims=True)
        idx = jnp.min(jnp.where(val == m, col, NS), axis=1, keepdims=True)
        onehot = col == idx
        val = jnp.where(onehot, NEG, val)
        mask = jnp.where(onehot, 1.0, mask)
        return val, mask

    _, mask = jax.lax.fori_loop(0, TOPK, step, (sim, jnp.zeros_like(sim)))
    mask_ref[...] = mask


def _v1_kernel(x_ref, emb_ref, wx_ref, we_ref, out_ref, *, nb):
    ev = jnp.dot(emb_ref[...], we_ref[...], preferred_element_type=jnp.float32)
    for b in range(nb):
        vb = jnp.dot(x_ref[b * NS:(b + 1) * NS, :], wx_ref[...],
                     preferred_element_type=jnp.float32) + ev
        for h in range(HEADS):
            out_ref[h, :, b * DH:(b + 1) * DH] = vb[:, h * DH:(h + 1) * DH]


def _masked_alpha(qa_h, kb_h, mask):
    e = qa_h + kb_h
    e = jnp.where(e >= 0, e, 0.2 * e)
    e = jnp.where(mask > 0, e, NEG)
    m = jnp.max(e, axis=1, keepdims=True)
    p = jnp.where(mask > 0, jnp.exp(e - m), 0.0)
    s = jnp.sum(p, axis=1, keepdims=True)
    return p / (s + 1e-8)


def _layer1_kernel(mask_ref, emb_ref, wqa_ref, wkb_ref, v1_ref, out_ref, *, nb):
    i = pl.program_id(0)
    emb = emb_ref[...]
    rows = emb_ref[pl.ds(i * RB, RB), :]
    qa = jnp.dot(rows, wqa_ref[...], preferred_element_type=jnp.float32)
    kbT = jax.lax.dot_general(wkb_ref[...], emb, (((0,), (1,)), ((), ())),
                              preferred_element_type=jnp.float32)
    mask = mask_ref[...]
    for h in range(HEADS):
        alpha = _masked_alpha(qa[:, h:h + 1], kbT[h:h + 1, :], mask)
        zh = jnp.dot(alpha, v1_ref[h], preferred_element_type=jnp.float32)
        for b in range(nb):
            out_ref[b, :, h * DH:(h + 1) * DH] = zh[:, b * DH:(b + 1) * DH]


def _elu(z):
    return jnp.where(z > 0, z, jnp.exp(z) - 1.0)


def _ln(z, g, b):
    mu = jnp.mean(z, axis=1, keepdims=True)
    var = jnp.mean((z - mu) ** 2, axis=1, keepdims=True)
    return (z - mu) / jnp.sqrt(var + 1e-5) * g + b


def _mid_kernel(z_ref, emb_ref, wz_ref, we_ref, g_ref, b_ref, out_ref, *, nb):
    ev = jnp.dot(emb_ref[...], we_ref[...], preferred_element_type=jnp.float32)
    for b in range(nb):
        zn = _ln(_elu(z_ref[b]), g_ref[...], b_ref[...])
        out_ref[:, b * HID:(b + 1) * HID] = (
            jnp.dot(zn, wz_ref[...], preferred_element_type=jnp.float32) + ev)


def _layer2_kernel(mask_ref, emb_ref, wqa_ref, wkb_ref, v2_ref, g_ref, b_ref,
                   wf1z_ref, wf1e_ref, bf1_ref, wf2_ref, bf2_ref, out_ref, *, nb):
    i = pl.program_id(0)
    emb = emb_ref[...]
    rows = emb_ref[pl.ds(i * RB, RB), :]
    qa = jnp.dot(rows, wqa_ref[...], preferred_element_type=jnp.float32)
    kbT = jax.lax.dot_general(wkb_ref[...], emb, (((0,), (1,)), ((), ())),
                              preferred_element_type=jnp.float32)
    alpha = _masked_alpha(qa, kbT, mask_ref[...])
    z2 = jnp.dot(alpha, v2_ref[...], preferred_element_type=jnp.float32)
    embf = jnp.dot(rows, wf1e_ref[...], preferred_element_type=jnp.float32)
    embf = embf + bf1_ref[...]
    for b in range(nb):
        zb = _ln(_elu(z2[:, b * HID:(b + 1) * HID]), g_ref[...], b_ref[...])
        h1 = jnp.dot(zb, wf1z_ref[...], preferred_element_type=jnp.float32) + embf
        h1 = jnp.maximum(h1, 0.0)
        p = jnp.dot(h1, wf2_ref[...], preferred_element_type=jnp.float32)
        out_ref[:, b:b + 1] = p + bf2_ref[...]


def kernel(x, batch, sensor_emb, Wq1, Wk1, Wv1, av1, ln1_g, ln1_b,
           Wq2, Wk2, Wv2, av2, ln2_g, ln2_b, Wf1, bf1, Wf2, bf2):
    nb = batch.shape[0] // NS
    f32 = jnp.float32

    eye = jnp.eye(HEADS, dtype=f32)
    avq1 = (av1[:, :DH][:, :, None] * eye[:, None, :]).reshape(HID, HEADS)
    avk1 = (av1[:, DH:][:, :, None] * eye[:, None, :]).reshape(HID, HEADS)
    Wqa1 = Wq1 @ avq1
    Wkb1 = Wk1 @ avk1
    Wqa2 = Wq2 @ av2[0, :HID][:, None]
    Wkb2 = Wk2 @ av2[0, HID:][:, None]

    mask = pl.pallas_call(
        _topk_mask_kernel,
        grid=(NBLK,),
        in_specs=[pl.BlockSpec((NS, HID), lambda i: (0, 0))],
        out_specs=pl.BlockSpec((RB, NS), lambda i: (i, 0)),
        out_shape=jax.ShapeDtypeStruct((NS, NS), f32),
    )(sensor_emb)

    v1 = pl.pallas_call(
        functools.partial(_v1_kernel, nb=nb),
        in_specs=[
            pl.BlockSpec((nb * NS, IN_DIM), lambda: (0, 0)),
            pl.BlockSpec((NS, HID), lambda: (0, 0)),
            pl.BlockSpec((IN_DIM, HID), lambda: (0, 0)),
            pl.BlockSpec((HID, HID), lambda: (0, 0)),
        ],
        out_specs=pl.BlockSpec((HEADS, NS, nb * DH), lambda: (0, 0, 0)),
        out_shape=jax.ShapeDtypeStruct((HEADS, NS, nb * DH), f32),
    )(x, sensor_emb, Wv1[:IN_DIM], Wv1[IN_DIM:])

    z1 = pl.pallas_call(
        functools.partial(_layer1_kernel, nb=nb),
        grid=(NBLK,),
        in_specs=[
            pl.BlockSpec((RB, NS), lambda i: (i, 0)),
            pl.BlockSpec((NS, HID), lambda i: (0, 0)),
            pl.BlockSpec((HID, HEADS), lambda i: (0, 0)),
            pl.BlockSpec((HID, HEADS), lambda i: (0, 0)),
            pl.BlockSpec((HEADS, NS, nb * DH), lambda i: (0, 0, 0)),
        ],
        out_specs=pl.BlockSpec((nb, RB, HID), lambda i: (0, i, 0)),
        out_shape=jax.ShapeDtypeStruct((nb, NS, HID), f32),
    )(mask, sensor_emb, Wqa1, Wkb1, v1)

    v2 = pl.pallas_call(
        functools.partial(_mid_kernel, nb=nb),
        in_specs=[
            pl.BlockSpec((nb, NS, HID), lambda: (0, 0, 0)),
            pl.BlockSpec((NS, HID), lambda: (0, 0)),
            pl.BlockSpec((HID, HID), lambda: (0, 0)),
            pl.BlockSpec((HID, HID), lambda: (0, 0)),
            pl.BlockSpec((1, HID), lambda: (0, 0)),
            pl.BlockSpec((1, HID), lambda: (0, 0)),
        ],
        out_specs=pl.BlockSpec((NS, nb * HID), lambda: (0, 0)),
        out_shape=jax.ShapeDtypeStruct((NS, nb * HID), f32),
    )(z1, sensor_emb, Wv2[:HID], Wv2[HID:], ln1_g[None, :], ln1_b[None, :])

    pred = pl.pallas_call(
        functools.partial(_layer2_kernel, nb=nb),
        grid=(NBLK,),
        in_specs=[
            pl.BlockSpec((RB, NS), lambda i: (i, 0)),
            pl.BlockSpec((NS, HID), lambda i: (0, 0)),
            pl.BlockSpec((HID, 1), lambda i: (0, 0)),
            pl.BlockSpec((HID, 1), lambda i: (0, 0)),
            pl.BlockSpec((NS, nb * HID), lambda i: (0, 0)),
            pl.BlockSpec((1, HID), lambda i: (0, 0)),
            pl.BlockSpec((1, HID), lambda i: (0, 0)),
            pl.BlockSpec((HID, HID), lambda i: (0, 0)),
            pl.BlockSpec((HID, HID), lambda i: (0, 0)),
            pl.BlockSpec((1, HID), lambda i: (0, 0)),
            pl.BlockSpec((HID, 1), lambda i: (0, 0)),
            pl.BlockSpec((1, 1), lambda i: (0, 0)),
        ],
        out_specs=pl.BlockSpec((RB, nb), lambda i: (i, 0)),
        out_shape=jax.ShapeDtypeStruct((NS, nb), f32),
    )(mask, sensor_emb, Wqa2, Wkb2, v2, ln2_g[None, :], ln2_b[None, :],
      Wf1[:HID], Wf1[HID:], bf1[None, :], Wf2, bf2[None, :])

    return pred.T.reshape(-1)


# speedup vs baseline: 734.0237x; 1.6933x over previous
import functools

import jax
import jax.numpy as jnp
from jax.experimental import pallas as pl

NS = 2048
IN_DIM = 64
HID = 64
TOPK = 32
HEADS = 4
DH = HID // HEADS
RB = 256
NBLK = NS // RB
NEG = -1e30
IMIN = -2**31
IMAX = 2**31 - 1


def _sortable_keys(sim):
    b = jax.lax.bitcast_convert_type(sim, jnp.int32)
    return jnp.where(b >= 0, b,
                     jnp.bitwise_xor(jnp.bitwise_not(b), jnp.int32(IMIN)))


def _topk_mask(sim):
    k = _sortable_keys(sim)
    kf = TOPK * 1.0

    def step(_, carry):
        lo, hi = carry
        mid = (lo & hi) + ((lo ^ hi) >> 1)
        cnt = jnp.sum(jnp.where(k >= mid, 1.0, 0.0), axis=1, keepdims=True)
        ge = cnt >= kf
        return jnp.where(ge, mid, lo), jnp.where(ge, hi, mid)

    lo0 = jnp.full((sim.shape[0], 1), IMIN, jnp.int32)
    hi0 = jnp.full((sim.shape[0], 1), IMAX, jnp.int32)
    t, _ = jax.lax.fori_loop(0, 32, step, (lo0, hi0))
    gtf = jnp.where(k > t, 1.0, 0.0)
    eqf = jnp.where(k == t, 1.0, 0.0)
    c1 = jnp.sum(gtf, axis=1, keepdims=True)
    cum = eqf
    s = 1
    while s < NS:
        cum = cum + jnp.concatenate(
            [jnp.zeros((sim.shape[0], s), jnp.float32), cum[:, :NS - s]], axis=1)
        s *= 2
    return gtf + eqf * jnp.where(c1 + cum <= kf, 1.0, 0.0)


def _v1_kernel(x_ref, emb_ref, wx_ref, we_ref, out_ref, *, nb):
    ev = jnp.dot(emb_ref[...], we_ref[...], preferred_element_type=jnp.float32)
    for b in range(nb):
        vb = jnp.dot(x_ref[b * NS:(b + 1) * NS, :], wx_ref[...],
                     preferred_element_type=jnp.float32) + ev
        for h in range(HEADS):
            out_ref[h, :, b * DH:(b + 1) * DH] = vb[:, h * DH:(h + 1) * DH]


def _masked_alpha(qa_h, kb_h, mask):
    e = qa_h + kb_h
    e = jnp.where(e >= 0, e, 0.2 * e)
    e = jnp.where(mask > 0, e, NEG)
    m = jnp.max(e, axis=1, keepdims=True)
    p = jnp.where(mask > 0, jnp.exp(e - m), 0.0)
    s = jnp.sum(p, axis=1, keepdims=True)
    return p / (s + 1e-8)


def _elu(z):
    return jnp.where(z > 0, z, jnp.exp(z) - 1.0)


def _ln(z, g, b):
    mu = jnp.mean(z, axis=1, keepdims=True)
    var = jnp.mean((z - mu) ** 2, axis=1, keepdims=True)
    return (z - mu) / jnp.sqrt(var + 1e-5) * g + b


def _layer1_kernel(emb_ref, wqa_ref, wkb_ref, v1_ref, wv2z_ref, wv2e_ref,
                   g_ref, b_ref, mask_ref, v2_ref, *, nb):
    i = pl.program_id(0)
    emb = emb_ref[...]
    nrm = jnp.sqrt(jnp.sum(emb * emb, axis=1, keepdims=True))
    embn = emb / jnp.maximum(nrm, 1e-12)
    rows_raw = emb_ref[pl.ds(i * RB, RB), :]
    rnrm = jnp.sqrt(jnp.sum(rows_raw * rows_raw, axis=1, keepdims=True))
    rows = rows_raw / jnp.maximum(rnrm, 1e-12)
    sim = jax.lax.dot_general(rows, embn, (((1,), (1,)), ((), ())),
                              preferred_element_type=jnp.float32)
    col = jax.lax.broadcasted_iota(jnp.int32, (RB, NS), 1)
    row = jax.lax.broadcasted_iota(jnp.int32, (RB, NS), 0)
    sim = jnp.where(col == row + i * RB, -1e9, sim)
    mask = _topk_mask(sim)
    mask_ref[...] = mask.astype(jnp.bfloat16)

    qa = jnp.dot(rows_raw, wqa_ref[...], preferred_element_type=jnp.float32)
    kbT = jax.lax.dot_general(wkb_ref[...], emb, (((0,), (1,)), ((), ())),
                              preferred_element_type=jnp.float32)
    zh = []
    for h in range(HEADS):
        alpha = _masked_alpha(qa[:, h:h + 1], kbT[h:h + 1, :], mask)
        zh.append(jnp.dot(alpha, v1_ref[h], preferred_element_type=jnp.float32))

    embv2 = jnp.dot(rows_raw, wv2e_ref[...], preferred_element_type=jnp.float32)
    for b in range(nb):
        zb = jnp.concatenate([zh[h][:, b * DH:(b + 1) * DH]
                              for h in range(HEADS)], axis=1)
        zb = _ln(_elu(zb), g_ref[...], b_ref[...])
        v2b = jnp.dot(zb, wv2z_ref[...], preferred_element_type=jnp.float32)
        v2_ref[:, b * HID:(b + 1) * HID] = v2b + embv2


def _layer2_kernel(mask_ref, emb_ref, wqa_ref, wkb_ref, v2_ref, g_ref, b_ref,
                   wf1z_ref, wf1e_ref, bf1_ref, wf2_ref, bf2_ref, out_ref, *, nb):
    i = pl.program_id(0)
    emb = emb_ref[...]
    rows = emb_ref[pl.ds(i * RB, RB), :]
    qa = jnp.dot(rows, wqa_ref[...], preferred_element_type=jnp.float32)
    kbT = jax.lax.dot_general(wkb_ref[...], emb, (((0,), (1,)), ((), ())),
                              preferred_element_type=jnp.float32)
    alpha = _masked_alpha(qa, kbT, mask_ref[...])
    z2 = jnp.dot(alpha, v2_ref[...], preferred_element_type=jnp.float32)
    embf = jnp.dot(rows, wf1e_ref[...], preferred_element_type=jnp.float32)
    embf = embf + bf1_ref[...]
    for b in range(nb):
        zb = _ln(_elu(z2[:, b * HID:(b + 1) * HID]), g_ref[...], b_ref[...])
        h1 = jnp.dot(zb, wf1z_ref[...], preferred_element_type=jnp.float32) + embf
        h1 = jnp.maximum(h1, 0.0)
        p = jnp.dot(h1, wf2_ref[...], preferred_element_type=jnp.float32)
        out_ref[:, b:b + 1] = p + bf2_ref[...]


def kernel(x, batch, sensor_emb, Wq1, Wk1, Wv1, av1, ln1_g, ln1_b,
           Wq2, Wk2, Wv2, av2, ln2_g, ln2_b, Wf1, bf1, Wf2, bf2):
    nb = batch.shape[0] // NS
    f32 = jnp.float32

    eye = jnp.eye(HEADS, dtype=f32)
    avq1 = (av1[:, :DH][:, :, None] * eye[:, None, :]).reshape(HID, HEADS)
    avk1 = (av1[:, DH:][:, :, None] * eye[:, None, :]).reshape(HID, HEADS)
    Wqa1 = Wq1 @ avq1
    Wkb1 = Wk1 @ avk1
    Wqa2 = Wq2 @ av2[0, :HID][:, None]
    Wkb2 = Wk2 @ av2[0, HID:][:, None]

    v1 = pl.pallas_call(
        functools.partial(_v1_kernel, nb=nb),
        in_specs=[
            pl.BlockSpec((nb * NS, IN_DIM), lambda: (0, 0)),
            pl.BlockSpec((NS, HID), lambda: (0, 0)),
            pl.BlockSpec((IN_DIM, HID), lambda: (0, 0)),
            pl.BlockSpec((HID, HID), lambda: (0, 0)),
        ],
        out_specs=pl.BlockSpec((HEADS, NS, nb * DH), lambda: (0, 0, 0)),
        out_shape=jax.ShapeDtypeStruct((HEADS, NS, nb * DH), f32),
    )(x, sensor_emb, Wv1[:IN_DIM], Wv1[IN_DIM:])

    mask, v2 = pl.pallas_call(
        functools.partial(_layer1_kernel, nb=nb),
        grid=(NBLK,),
        in_specs=[
            pl.BlockSpec((NS, HID), lambda i: (0, 0)),
            pl.BlockSpec((HID, HEADS), lambda i: (0, 0)),
            pl.BlockSpec((HID, HEADS), lambda i: (0, 0)),
            pl.BlockSpec((HEADS, NS, nb * DH), lambda i: (0, 0, 0)),
            pl.BlockSpec((HID, HID), lambda i: (0, 0)),
            pl.BlockSpec((HID, HID), lambda i: (0, 0)),
            pl.BlockSpec((1, HID), lambda i: (0, 0)),
            pl.BlockSpec((1, HID), lambda i: (0, 0)),
        ],
        out_specs=[
            pl.BlockSpec((RB, NS), lambda i: (i, 0)),
            pl.BlockSpec((RB, nb * HID), lambda i: (i, 0)),
        ],
        out_shape=[
            jax.ShapeDtypeStruct((NS, NS), jnp.bfloat16),
            jax.ShapeDtypeStruct((NS, nb * HID), f32),
        ],
    )(sensor_emb, Wqa1, Wkb1, v1, Wv2[:HID], Wv2[HID:],
      ln1_g[None, :], ln1_b[None, :])

    pred = pl.pallas_call(
        functools.partial(_layer2_kernel, nb=nb),
        grid=(NBLK,),
        in_specs=[
            pl.BlockSpec((RB, NS), lambda i: (i, 0)),
            pl.BlockSpec((NS, HID), lambda i: (0, 0)),
            pl.BlockSpec((HID, 1), lambda i: (0, 0)),
            pl.BlockSpec((HID, 1), lambda i: (0, 0)),
            pl.BlockSpec((NS, nb * HID), lambda i: (0, 0)),
            pl.BlockSpec((1, HID), lambda i: (0, 0)),
            pl.BlockSpec((1, HID), lambda i: (0, 0)),
            pl.BlockSpec((HID, HID), lambda i: (0, 0)),
            pl.BlockSpec((HID, HID), lambda i: (0, 0)),
            pl.BlockSpec((1, HID), lambda i: (0, 0)),
            pl.BlockSpec((HID, 1), lambda i: (0, 0)),
            pl.BlockSpec((1, 1), lambda i: (0, 0)),
        ],
        out_specs=pl.BlockSpec((RB, nb), lambda i: (i, 0)),
        out_shape=jax.ShapeDtypeStruct((NS, nb), f32),
    )(mask, sensor_emb, Wqa2, Wkb2, v2, ln2_g[None, :], ln2_b[None, :],
      Wf1[:HID], Wf1[HID:], bf1[None, :], Wf2, bf2[None, :])

    return pred.T.reshape(-1)
